# baseline (speedup 1.0000x reference)
import jax
import jax.numpy as jnp
from jax.experimental import pallas as pl
from jax.experimental.pallas import tpu as pltpu

_CAPACITY = 28
_NB = 8


def _router_kernel(*refs):
    x_refs = refs[:_NB]
    w_ref, mask_ref, weights_ref, ratio_ref = refs[_NB:]
    g = pl.program_id(0)
    nsteps = pl.num_programs(0)

    _, S, D = x_refs[0].shape
    w = w_ref[...]
    E = w.shape[0]
    C = _CAPACITY

    s_idx = jax.lax.broadcasted_iota(jnp.int32, (S, 1, 1), 0)
    t_idx = jax.lax.broadcasted_iota(jnp.int32, (1, 1, S), 2)
    tie = t_idx < s_idx

    j_e = jax.lax.broadcasted_iota(jnp.int32, (E, E * C), 1) // C
    e_i = jax.lax.broadcasted_iota(jnp.int32, (E, E * C), 0)
    P = (j_e == e_i).astype(jnp.float32)
    cmod = (jax.lax.broadcasted_iota(jnp.int32, (1, E * C), 1) % C
            ).astype(jnp.float32)

    gates_l = []
    rank_l = []
    frac = jnp.zeros((1, 1), jnp.float32)
    for i in range(_NB):
        x = x_refs[i][0]
        logits = jax.lax.dot_general(
            x, w, (((1,), (1,)), ((), ())),
            preferred_element_type=jnp.float32)
        m = jnp.max(logits, axis=1, keepdims=True)
        ex = jnp.exp(logits - m)
        gates = ex / jnp.sum(ex, axis=1, keepdims=True)

        ga = gates[:, :, None]
        gb = jnp.transpose(gates)[None, :, :]
        beats = (gb > ga) | ((gb == ga) & tie)
        rank = jnp.sum(beats.astype(jnp.float32), axis=2)

        gates_l.append(gates)
        rank_l.append(rank)
        processed = (jnp.min(rank, axis=1, keepdims=True) < C)
        frac += (jnp.sum(processed.astype(jnp.float32), axis=0, keepdims=True)
                 * (1.0 / (S * _NB * nsteps)))

    for i in range(_NB):
        rank_rep = jax.lax.dot_general(
            rank_l[i], P, (((1,), (0,)), ((), ())),
            preferred_element_type=jnp.float32)
        gates_rep = jax.lax.dot_general(
            gates_l[i], P, (((1,), (0,)), ((), ())),
            preferred_element_type=jnp.float32)
        mask2 = (rank_rep == cmod).astype(jnp.float32)
        mask_ref[i] = mask2
        weights_ref[i] = mask2 * gates_rep

    @pl.when(g == 0)
    def _init():
        ratio_ref[...] = frac

    @pl.when(g != 0)
    def _acc():
        ratio_ref[...] += frac


def kernel(inputs, W):
    B, S, D = inputs.shape
    E = W.shape[0]
    C = _CAPACITY
    NB = _NB

    x_specs = [
        pl.BlockSpec((1, S, D), lambda g, i=i: (g * NB + i, 0, 0))
        for i in range(NB)
    ]
    mask_flat, weights_flat, ratio = pl.pallas_call(
        _router_kernel,
        grid=(B // NB,),
        in_specs=x_specs + [
            pl.BlockSpec((E, D), lambda g: (0, 0)),
        ],
        out_specs=[
            pl.BlockSpec((NB, S, E * C), lambda g: (g, 0, 0)),
            pl.BlockSpec((NB, S, E * C), lambda g: (g, 0, 0)),
            pl.BlockSpec((1, 1), lambda g: (0, 0)),
        ],
        out_shape=[
            jax.ShapeDtypeStruct((B, S, E * C), jnp.float32),
            jax.ShapeDtypeStruct((B, S, E * C), jnp.float32),
            jax.ShapeDtypeStruct((1, 1), jnp.float32),
        ],
        compiler_params=pltpu.CompilerParams(
            vmem_limit_bytes=120 * 1024 * 1024),
    )(*([inputs] * NB), W)

    mask = mask_flat.reshape(B, S, E, C)
    weights = weights_flat.reshape(B, S, E, C)
    ratio_processed_items = ratio[0, 0]
    auxiliary_loss = jnp.array(0.0, dtype=jnp.float32)
    return mask, weights, ratio_processed_items, auxiliary_loss
